# baseline (speedup 1.0000x reference)
import functools

import jax
import jax.numpy as jnp
from jax import lax
from jax.experimental import pallas as pl
from jax.experimental.pallas import tpu as pltpu
from jax.experimental.pallas import tpu_sc as plsc

F32 = jnp.float32
BF16 = jnp.bfloat16

N = 10000
E = 320000
H = 128
NH = 8
DH = 16
INNER = 512

NB = 1000
EB = 2560
NC = 2
NS = 16
NW = NC * NS
EPW = E // NW
CH = 80
NCHUNK = EPW // CH
NPAIR = NCHUNK // 2

RN = 16
NRCHUNK = N // RN
RITER = (NRCHUNK + NW - 1) // NW

_mesh = plsc.VectorSubcoreMesh(core_axis_name="c", subcore_axis_name="s")
_sc_params = pltpu.CompilerParams(needs_layout_passes=False,
                                  use_tc_tiling_on_sc=False)


def _bdot(a, w_ref):
    return jnp.dot(a.astype(BF16), w_ref[...], preferred_element_type=F32)


def _ln(v):
    mu = jnp.mean(v, axis=-1, keepdims=True)
    var = jnp.mean((v - mu) ** 2, axis=-1, keepdims=True)
    return (v - mu) * lax.rsqrt(var + 1e-6)


def _k1_body(x_ref, te_ref, wada_ref, bada_ref, wqkv_ref, qkv_ref, mod_ref):
    mod = _bdot(jax.nn.silu(te_ref[...]), wada_ref) + bada_ref[...]
    mod_ref[...] = mod
    xm = _ln(x_ref[...]) * (1.0 + mod[:, 128:256]) + mod[:, 0:128]
    qkv_ref[...] = _bdot(xm, wqkv_ref)


def _k1(x, t_emb_h, W_ada, b_ada, W_qkv):
    return pl.pallas_call(
        _k1_body,
        grid=(N // NB,),
        in_specs=[
            pl.BlockSpec((NB, H), lambda i: (i, 0)),
            pl.BlockSpec((NB, H), lambda i: (i, 0)),
            pl.BlockSpec((H, 6 * H), lambda i: (0, 0)),
            pl.BlockSpec((1, 6 * H), lambda i: (0, 0)),
            pl.BlockSpec((H, 3 * H), lambda i: (0, 0)),
        ],
        out_specs=[
            pl.BlockSpec((NB, 3 * H), lambda i: (i, 0)),
            pl.BlockSpec((NB, 6 * H), lambda i: (i, 0)),
        ],
        out_shape=[
            jax.ShapeDtypeStruct((N, 3 * H), F32),
            jax.ShapeDtypeStruct((N, 6 * H), F32),
        ],
    )(x, t_emb_h, W_ada, b_ada, W_qkv)


def _k2_body(eattr_ref, te_ref, dist_ref, wee1_ref, wee2_ref, bee_ref,
             wade_ref, bade_ref, wle0_ref, wle1_ref,
             ea_ref, eattn_ref, le1_ref):
    mod2 = _bdot(jax.nn.silu(te_ref[...]), wade_ref) + bade_ref[...]
    ea = (_bdot(eattr_ref[...], wee1_ref) + _bdot(dist_ref[...], wee2_ref)
          + bee_ref[...])
    ea_ref[...] = ea.astype(BF16)
    eam = _ln(ea) * (1.0 + mod2[:, 128:256]) + mod2[:, 0:128]
    eattn_ref[...] = jax.nn.gelu(_bdot(eam, wle0_ref), approximate=True) * 0.25
    le1_ref[...] = _bdot(eam, wle1_ref)


def _k2(edge_attr, t_emb_e, dist, Wee1, Wee2, bee, Wade2, bade2, W_le0, W_le1):
    full = lambda a, b: pl.BlockSpec((a, b), lambda i: (0, 0))
    row = lambda w: pl.BlockSpec((EB, w), lambda i: (i, 0))
    return pl.pallas_call(
        _k2_body,
        grid=(E // EB,),
        in_specs=[row(H), row(H), row(H),
                  full(H, H), full(H, H), full(1, H),
                  full(H, 2 * H), full(1, 2 * H), full(H, H), full(H, H)],
        out_specs=[row(H), row(H), row(H)],
        out_shape=[jax.ShapeDtypeStruct((E, H), BF16),
                   jax.ShapeDtypeStruct((E, H), F32),
                   jax.ShapeDtypeStruct((E, H), F32)],
    )(edge_attr, t_emb_e, dist, Wee1, Wee2, bee, Wade2, bade2, W_le0, W_le1)


@functools.partial(
    pl.kernel,
    out_type=(jax.ShapeDtypeStruct((E, 16), F32),
              jax.ShapeDtypeStruct((NC, N, 16), F32)),
    mesh=_mesh,
    compiler_params=_sc_params,
    scratch_types=[
        pltpu.VMEM((NCHUNK, CH), jnp.int32),
        pltpu.VMEM((NCHUNK, CH), jnp.int32),
        pltpu.VMEM((2, CH, H), F32),
        pltpu.VMEM((2, CH, H), F32),
        pltpu.VMEM((2, CH, H), F32),
        pltpu.VMEM((2, CH, 16), F32),
        pltpu.VMEM_SHARED((N, 16), F32),
        pltpu.SemaphoreType.DMA,
        pltpu.SemaphoreType.DMA,
        pltpu.SemaphoreType.DMA,
        pltpu.SemaphoreType.DMA,
    ],
)
def _sca1(q_hbm, k_hbm, eattn_hbm, src3_hbm, tgt3_hbm, zs_hbm,
          expE_hbm, spart_hbm,
          src_v, tgt_v, qbuf, kbuf, abuf, ebuf, accs,
          g0, g1, o0, o1):
    c = lax.axis_index("c")
    s = lax.axis_index("s")
    wid = s * NC + c

    @pl.when(s == 0)
    def _():
        pltpu.sync_copy(zs_hbm, accs)

    pltpu.sync_copy(src3_hbm.at[wid], src_v)
    pltpu.sync_copy(tgt3_hbm.at[wid], tgt_v)
    plsc.subcore_barrier()
    base0 = wid * EPW
    lane = lax.iota(jnp.int32, 16)
    gsems = (g0, g1)
    osems = (o0, o1)

    def issue(i, b):
        base = base0 + i * CH
        pltpu.async_copy(q_hbm.at[src_v.at[i]], qbuf.at[b], gsems[b])
        pltpu.async_copy(k_hbm.at[tgt_v.at[i]], kbuf.at[b], gsems[b])
        pltpu.async_copy(eattn_hbm.at[pl.ds(base, CH)], abuf.at[b], gsems[b])

    def wait_in(b):
        pltpu.make_async_copy(q_hbm.at[src_v.at[0]], qbuf.at[b], gsems[b]).wait()
        pltpu.make_async_copy(k_hbm.at[tgt_v.at[0]], kbuf.at[b], gsems[b]).wait()
        pltpu.make_async_copy(eattn_hbm.at[pl.ds(0, CH)], abuf.at[b], gsems[b]).wait()

    def compute(i, b):
        def edge_body(e, _):
            parts = []
            for h in range(NH):
                pv = (qbuf[b, e, pl.ds(16 * h, 16)]
                      * kbuf[b, e, pl.ds(16 * h, 16)]
                      * abuf[b, e, pl.ds(16 * h, 16)])
                parts.append(jnp.where(lane == h, jnp.sum(pv), 0.0))
            while len(parts) > 1:
                parts = [parts[i] + parts[i + 1]
                         for i in range(0, len(parts), 2)]
            ebuf[b, e, :] = jnp.exp(parts[0])
            return 0

        lax.fori_loop(0, CH, edge_body, 0, unroll=4)
        base = base0 + i * CH
        pltpu.async_copy(ebuf.at[b], expE_hbm.at[pl.ds(base, CH)], osems[b])
        pltpu.sync_copy(ebuf.at[b], accs.at[tgt_v.at[i]], add=True)

    def wait_out(b):
        pltpu.make_async_copy(ebuf.at[b], expE_hbm.at[pl.ds(0, CH)], osems[b]).wait()

    issue(0, 0)
    issue(1, 1)

    def pair_body(j, _):
        i0 = 2 * j
        wait_in(0)

        @pl.when(j > 0)
        def _():
            wait_out(0)

        compute(i0, 0)

        @pl.when(i0 + 2 < NCHUNK)
        def _():
            issue(i0 + 2, 0)

        wait_in(1)

        @pl.when(j > 0)
        def _():
            wait_out(1)

        compute(i0 + 1, 1)

        @pl.when(i0 + 3 < NCHUNK)
        def _():
            issue(i0 + 3, 1)

        return 0

    lax.fori_loop(0, NPAIR, pair_body, 0)
    wait_in(0)
    wait_out(0)
    compute(NCHUNK - 1, 0)
    wait_out(0)
    wait_out(1)
    plsc.subcore_barrier()

    @pl.when(s == 0)
    def _():
        pltpu.sync_copy(accs, spart_hbm.at[c])


@functools.partial(
    pl.kernel,
    out_type=jax.ShapeDtypeStruct((NC, N, H), F32),
    mesh=_mesh,
    compiler_params=_sc_params,
    scratch_types=[
        pltpu.VMEM((NCHUNK, CH), jnp.int32),
        pltpu.VMEM((2, CH, H), F32),
        pltpu.VMEM((2, CH, 16), F32),
        pltpu.VMEM_SHARED((N, H), F32),
        pltpu.SemaphoreType.DMA,
        pltpu.SemaphoreType.DMA,
        pltpu.SemaphoreType.DMA,
        pltpu.SemaphoreType.DMA,
    ],
)
def _sca2(le1_hbm, expE_hbm, tgt3_hbm, zm_hbm,
          mpart_hbm,
          tgt_v, lbuf, ebuf, accm,
          l0, l1, o0, o1):
    c = lax.axis_index("c")
    s = lax.axis_index("s")
    wid = s * NC + c

    @pl.when(s == 0)
    def _():
        pltpu.sync_copy(zm_hbm, accm)

    pltpu.sync_copy(tgt3_hbm.at[wid], tgt_v)
    plsc.subcore_barrier()
    base0 = wid * EPW
    lsems = (l0, l1)
    osems = (o0, o1)

    def issue(i, b):
        base = base0 + i * CH
        pltpu.async_copy(le1_hbm.at[pl.ds(base, CH)], lbuf.at[b], lsems[b])
        pltpu.async_copy(expE_hbm.at[pl.ds(base, CH)], ebuf.at[b], lsems[b])

    def wait_in(b):
        pltpu.make_async_copy(le1_hbm.at[pl.ds(0, CH)], lbuf.at[b], lsems[b]).wait()
        pltpu.make_async_copy(expE_hbm.at[pl.ds(0, CH)], ebuf.at[b], lsems[b]).wait()

    def compute(i, b):
        def edge_body(e, _):
            ev = ebuf[b, e, :]
            for h in range(NH):
                lbuf[b, e, pl.ds(16 * h, 16)] = (
                    lbuf[b, e, pl.ds(16 * h, 16)] * ev[h])
            return 0

        lax.fori_loop(0, CH, edge_body, 0, unroll=4)
        pltpu.sync_copy(lbuf.at[b], accm.at[tgt_v.at[i]], add=True)

    def wait_out(b):
        pass

    issue(0, 0)
    issue(1, 1)

    def pair_body(j, _):
        i0 = 2 * j
        wait_in(0)

        @pl.when(j > 0)
        def _():
            wait_out(0)

        compute(i0, 0)

        @pl.when(i0 + 2 < NCHUNK)
        def _():
            issue(i0 + 2, 0)

        wait_in(1)

        @pl.when(j > 0)
        def _():
            wait_out(1)

        compute(i0 + 1, 1)

        @pl.when(i0 + 3 < NCHUNK)
        def _():
            issue(i0 + 3, 1)

        return 0

    lax.fori_loop(0, NPAIR, pair_body, 0)
    wait_in(0)
    wait_out(0)
    compute(NCHUNK - 1, 0)
    wait_out(0)
    wait_out(1)
    plsc.subcore_barrier()

    @pl.when(s == 0)
    def _():
        pltpu.sync_copy(accm, mpart_hbm.at[c])


@functools.partial(
    pl.kernel,
    out_type=jax.ShapeDtypeStruct((N, H), F32),
    mesh=_mesh,
    compiler_params=_sc_params,
    scratch_types=[
        pltpu.VMEM((RN, H), F32),
        pltpu.VMEM((RN, H), F32),
        pltpu.VMEM((RN, H), F32),
        pltpu.VMEM((RN, 16), F32),
        pltpu.VMEM((RN, 16), F32),
        pltpu.SemaphoreType.DMA,
        pltpu.SemaphoreType.DMA,
        pltpu.SemaphoreType.DMA,
        pltpu.SemaphoreType.DMA,
    ],
)
def _scn(v_hbm, mpart_hbm, spart_hbm, out_hbm,
         m0buf, m1buf, vbuf, s0buf, s1buf, sem1, sem2, sem3, sem4):
    c = lax.axis_index("c")
    s = lax.axis_index("s")
    wid = s * NC + c

    def rchunk_body(i, _):
        cid = wid + i * NW

        @pl.when(cid < NRCHUNK)
        def _():
            base = cid * RN
            c0 = pltpu.async_copy(mpart_hbm.at[0].at[pl.ds(base, RN)], m0buf, sem1)
            c1 = pltpu.async_copy(mpart_hbm.at[1].at[pl.ds(base, RN)], m1buf, sem2)
            c2 = pltpu.async_copy(spart_hbm.at[0].at[pl.ds(base, RN)], s0buf, sem3)
            c3 = pltpu.async_copy(v_hbm.at[pl.ds(base, RN)], vbuf, sem4)
            pltpu.sync_copy(spart_hbm.at[1].at[pl.ds(base, RN)], s1buf)
            c0.wait()
            c1.wait()
            c2.wait()
            c3.wait()

            def row_body(r, _):
                sv = s0buf[r, :] + s1buf[r, :]
                rec = 1.0 / (sv + 1e-16)
                for h in range(NH):
                    m0buf[r, pl.ds(16 * h, 16)] = (
                        (m0buf[r, pl.ds(16 * h, 16)]
                         + m1buf[r, pl.ds(16 * h, 16)])
                        * vbuf[r, pl.ds(16 * h, 16)] * rec[h])
                return 0

            lax.fori_loop(0, RN, row_body, 0)
            pltpu.sync_copy(m0buf, out_hbm.at[pl.ds(base, RN)])

        return 0

    lax.fori_loop(0, RITER, rchunk_body, 0)


def _k3_body(x_ref, out_ref, mod_ref, g2_ref, b2_ref,
             w1_ref, w3_ref, w2_ref, wn2e_ref, bn2e_ref,
             hout_ref, ow_ref):
    out = out_ref[...]
    mod = mod_ref[...]
    h = x_ref[...] + mod[:, 256:384] * out
    h = (_ln(h) * g2_ref[...] + b2_ref[...]) * (1.0 + mod[:, 512:640]) + mod[:, 384:512]
    sw = _bdot(jax.nn.silu(_bdot(h, w1_ref)) * _bdot(h, w3_ref), w2_ref)
    hout_ref[...] = h + mod[:, 640:768] * sw
    ow_ref[...] = (_bdot(out, wn2e_ref) + 0.5 * bn2e_ref[...]).astype(BF16)


def _k3(x, out, mod, g2, b2, W1, W3, W2, W_n2e, b_n2e):
    full = lambda a, b: pl.BlockSpec((a, b), lambda i: (0, 0))
    row = lambda w: pl.BlockSpec((NB, w), lambda i: (i, 0))
    return pl.pallas_call(
        _k3_body,
        grid=(N // NB,),
        in_specs=[row(H), row(H), row(6 * H),
                  full(1, H), full(1, H),
                  full(H, INNER), full(H, INNER), full(INNER, H),
                  full(H, H), full(1, H)],
        out_specs=[row(H), row(H)],
        out_shape=[jax.ShapeDtypeStruct((N, H), F32),
                   jax.ShapeDtypeStruct((N, H), BF16)],
    )(x, out, mod, g2, b2, W1, W3, W2, W_n2e, b_n2e)


@functools.partial(
    pl.kernel,
    out_type=jax.ShapeDtypeStruct((E, H), BF16),
    mesh=_mesh,
    compiler_params=_sc_params,
    scratch_types=[
        pltpu.VMEM((NCHUNK, CH), jnp.int32),
        pltpu.VMEM((NCHUNK, CH), jnp.int32),
        pltpu.VMEM((2, CH, H), BF16),
        pltpu.VMEM((2, CH, H), BF16),
        pltpu.SemaphoreType.DMA,
        pltpu.SemaphoreType.DMA,
        pltpu.SemaphoreType.DMA,
        pltpu.SemaphoreType.DMA,
    ],
)
def _sc3(ow_hbm, src3_hbm, tgt3_hbm, og_hbm,
         src_v, tgt_v, abuf, bbuf, g0, g1, o0, o1):
    c = lax.axis_index("c")
    s = lax.axis_index("s")
    wid = s * NC + c
    base0 = wid * EPW
    pltpu.sync_copy(src3_hbm.at[wid], src_v)
    pltpu.sync_copy(tgt3_hbm.at[wid], tgt_v)
    gsems = (g0, g1)
    osems = (o0, o1)

    def issue(i, b):
        pltpu.async_copy(ow_hbm.at[src_v.at[i]], abuf.at[b], gsems[b])
        pltpu.async_copy(ow_hbm.at[tgt_v.at[i]], bbuf.at[b], gsems[b])

    def wait_in(b):
        pltpu.make_async_copy(ow_hbm.at[src_v.at[0]], abuf.at[b], gsems[b]).wait()
        pltpu.make_async_copy(ow_hbm.at[tgt_v.at[0]], bbuf.at[b], gsems[b]).wait()

    def compute(i, b):
        def edge_body(e, _):
            for g in range(4):
                abuf[b, e, pl.ds(32 * g, 32)] = (
                    abuf[b, e, pl.ds(32 * g, 32)]
                    + bbuf[b, e, pl.ds(32 * g, 32)])
            return 0

        lax.fori_loop(0, CH, edge_body, 0, unroll=4)
        base = base0 + i * CH
        pltpu.async_copy(abuf.at[b], og_hbm.at[pl.ds(base, CH)], osems[b])

    def wait_out(b):
        pltpu.make_async_copy(abuf.at[b], og_hbm.at[pl.ds(0, CH)], osems[b]).wait()

    issue(0, 0)
    issue(1, 1)

    def pair_body(j, _):
        i0 = 2 * j
        wait_in(0)

        @pl.when(j > 0)
        def _():
            wait_out(0)

        compute(i0, 0)

        @pl.when(i0 + 2 < NCHUNK)
        def _():
            issue(i0 + 2, 0)

        wait_in(1)

        @pl.when(j > 0)
        def _():
            wait_out(1)

        compute(i0 + 1, 1)

        @pl.when(i0 + 3 < NCHUNK)
        def _():
            issue(i0 + 3, 1)

        return 0

    lax.fori_loop(0, NPAIR, pair_body, 0)
    wait_in(0)
    wait_out(0)
    compute(NCHUNK - 1, 0)
    wait_out(0)
    wait_out(1)


def _k4_body(og_ref, eattr_ref, te_ref, ea_ref, wade_ref, bade_ref,
             we1_ref, we3_ref, we2_ref, out_ref):
    mod4 = _bdot(jax.nn.silu(te_ref[...]), wade_ref) + bade_ref[...]
    h = eattr_ref[...] + mod4[:, 0:128] * og_ref[...].astype(F32)
    h = _ln(h) * (1.0 + mod4[:, 256:384]) + mod4[:, 128:256]
    sw = _bdot(jax.nn.silu(_bdot(h, we1_ref)) * _bdot(h, we3_ref), we2_ref)
    out_ref[...] = ea_ref[...].astype(F32) + h + mod4[:, 384:512] * sw


def _k4(og, edge_attr, t_emb_e, ea, Wade4, bade4, We1, We3, We2):
    full = lambda a, b: pl.BlockSpec((a, b), lambda i: (0, 0))
    row = lambda w: pl.BlockSpec((EB, w), lambda i: (i, 0))
    return pl.pallas_call(
        _k4_body,
        grid=(E // EB,),
        in_specs=[row(H), row(H), row(H), row(H),
                  full(H, 4 * H), full(1, 4 * H),
                  full(H, INNER), full(H, INNER), full(INNER, H)],
        out_specs=row(H),
        out_shape=jax.ShapeDtypeStruct((E, H), F32),
    )(og, edge_attr, t_emb_e, ea, Wade4, bade4, We1, We3, We2)


def kernel(batch, x, t_emb_h, edge_attr, edge_index, t_emb_e, dist,
           W_edge_emb, b_edge_emb, W_ada, b_ada, W_ada_e, b_ada_e,
           W_qkv, W_le0, W_le1, W_n2e, b_n2e, g2, b2,
           W1, W3, W2, We1, We3, We2):
    src3 = edge_index[0].reshape(NW, NCHUNK, CH)
    tgt3 = edge_index[1].reshape(NW, NCHUNK, CH)

    bf = lambda w: w.astype(BF16)
    qkv, mod = _k1(x, t_emb_h, bf(W_ada), b_ada.reshape(1, -1), bf(W_qkv))
    Qn = qkv[:, 0:H]
    Kn = qkv[:, H:2 * H]
    Vn = qkv[:, 2 * H:3 * H]

    ea, e_attn, le1 = _k2(edge_attr, t_emb_e, dist,
                          bf(W_edge_emb[:H]), bf(W_edge_emb[H:]),
                          b_edge_emb.reshape(1, -1),
                          bf(W_ada_e[:, :2 * H]), b_ada_e[:2 * H].reshape(1, -1),
                          bf(W_le0), bf(W_le1))

    zs = jnp.zeros((N, 16), F32)
    expE, spart = _sca1(Qn, Kn, e_attn, src3, tgt3, zs)

    zm = jnp.zeros((N, H), F32)
    mpart = _sca2(le1, expE, tgt3, zm)

    out = _scn(Vn, mpart, spart)

    h_out, ow = _k3(x, out, mod,
                    g2.reshape(1, -1), b2.reshape(1, -1),
                    bf(W1), bf(W3), bf(W2), bf(W_n2e), b_n2e.reshape(1, -1))

    og = _sc3(ow, src3, tgt3)

    h_edge_out = _k4(og, edge_attr, t_emb_e, ea,
                     bf(W_ada_e[:, 2 * H:]), b_ada_e[2 * H:].reshape(1, -1),
                     bf(We1), bf(We3), bf(We2))

    return (h_out, h_edge_out)


# speedup vs baseline: 1.0882x; 1.0882x over previous
import functools

import jax
import jax.numpy as jnp
from jax import lax
from jax.experimental import pallas as pl
from jax.experimental.pallas import tpu as pltpu
from jax.experimental.pallas import tpu_sc as plsc

F32 = jnp.float32
BF16 = jnp.bfloat16

N = 10000
E = 320000
H = 128
NH = 8
DH = 16
INNER = 512

NB = 1000
EB = 2560
NC = 2
NS = 16
NW = NC * NS
EPW = E // NW
CH = 80
NCHUNK = EPW // CH
NPAIR = NCHUNK // 2

RN = 16
NRCHUNK = N // RN
RITER = (NRCHUNK + NW - 1) // NW

_mesh = plsc.VectorSubcoreMesh(core_axis_name="c", subcore_axis_name="s")
_sc_params = pltpu.CompilerParams(needs_layout_passes=False,
                                  use_tc_tiling_on_sc=False)


def _bdot(a, w_ref):
    return jnp.dot(a.astype(BF16), w_ref[...], preferred_element_type=F32)


def _ln(v):
    mu = jnp.mean(v, axis=-1, keepdims=True)
    var = jnp.mean((v - mu) ** 2, axis=-1, keepdims=True)
    return (v - mu) * lax.rsqrt(var + 1e-6)


def _k1_body(x_ref, te_ref, wada_ref, bada_ref, wqkv_ref, qkv_ref, mod_ref):
    mod = _bdot(jax.nn.silu(te_ref[...]), wada_ref) + bada_ref[...]
    mod_ref[...] = mod
    xm = _ln(x_ref[...]) * (1.0 + mod[:, 128:256]) + mod[:, 0:128]
    qkv_ref[...] = _bdot(xm, wqkv_ref)


def _k1(x, t_emb_h, W_ada, b_ada, W_qkv):
    return pl.pallas_call(
        _k1_body,
        grid=(N // NB,),
        in_specs=[
            pl.BlockSpec((NB, H), lambda i: (i, 0)),
            pl.BlockSpec((NB, H), lambda i: (i, 0)),
            pl.BlockSpec((H, 6 * H), lambda i: (0, 0)),
            pl.BlockSpec((1, 6 * H), lambda i: (0, 0)),
            pl.BlockSpec((H, 3 * H), lambda i: (0, 0)),
        ],
        out_specs=[
            pl.BlockSpec((NB, 3 * H), lambda i: (i, 0)),
            pl.BlockSpec((NB, 6 * H), lambda i: (i, 0)),
        ],
        out_shape=[
            jax.ShapeDtypeStruct((N, 3 * H), F32),
            jax.ShapeDtypeStruct((N, 6 * H), F32),
        ],
    )(x, t_emb_h, W_ada, b_ada, W_qkv)


def _k2_body(eattr_ref, te_ref, dist_ref, wee1_ref, wee2_ref, bee_ref,
             wade_ref, bade_ref, wle0_ref, wle1_ref,
             ea_ref, eattn_ref, le1_ref):
    mod2 = _bdot(jax.nn.silu(te_ref[...]), wade_ref) + bade_ref[...]
    ea = (_bdot(eattr_ref[...], wee1_ref) + _bdot(dist_ref[...], wee2_ref)
          + bee_ref[...])
    ea_ref[...] = ea.astype(BF16)
    eam = _ln(ea) * (1.0 + mod2[:, 128:256]) + mod2[:, 0:128]
    eattn_ref[...] = jax.nn.gelu(_bdot(eam, wle0_ref), approximate=True) * 0.25
    le1_ref[...] = _bdot(eam, wle1_ref)


def _k2(edge_attr, t_emb_e, dist, Wee1, Wee2, bee, Wade2, bade2, W_le0, W_le1):
    full = lambda a, b: pl.BlockSpec((a, b), lambda i: (0, 0))
    row = lambda w: pl.BlockSpec((EB, w), lambda i: (i, 0))
    return pl.pallas_call(
        _k2_body,
        grid=(E // EB,),
        in_specs=[row(H), row(H), row(H),
                  full(H, H), full(H, H), full(1, H),
                  full(H, 2 * H), full(1, 2 * H), full(H, H), full(H, H)],
        out_specs=[row(H), row(H), row(H)],
        out_shape=[jax.ShapeDtypeStruct((E, H), BF16),
                   jax.ShapeDtypeStruct((E, H), F32),
                   jax.ShapeDtypeStruct((E, H), F32)],
    )(edge_attr, t_emb_e, dist, Wee1, Wee2, bee, Wade2, bade2, W_le0, W_le1)


@functools.partial(
    pl.kernel,
    out_type=(jax.ShapeDtypeStruct((E, 16), F32),
              jax.ShapeDtypeStruct((NC, N, 16), F32)),
    mesh=_mesh,
    compiler_params=_sc_params,
    scratch_types=[
        pltpu.VMEM((NCHUNK, CH), jnp.int32),
        pltpu.VMEM((NCHUNK, CH), jnp.int32),
        pltpu.VMEM((2, CH, H), F32),
        pltpu.VMEM((2, CH, H), F32),
        pltpu.VMEM((2, CH, H), F32),
        pltpu.VMEM((2, CH, 16), F32),
        pltpu.VMEM_SHARED((N, 16), F32),
        pltpu.SemaphoreType.DMA,
        pltpu.SemaphoreType.DMA,
        pltpu.SemaphoreType.DMA,
        pltpu.SemaphoreType.DMA,
    ],
)
def _sca1(q_hbm, k_hbm, eattn_hbm, src3_hbm, tgt3_hbm, zs_hbm,
          expE_hbm, spart_hbm,
          src_v, tgt_v, qbuf, kbuf, abuf, ebuf, accs,
          g0, g1, o0, o1):
    c = lax.axis_index("c")
    s = lax.axis_index("s")
    wid = s * NC + c

    @pl.when(s == 0)
    def _():
        pltpu.sync_copy(zs_hbm, accs)

    pltpu.sync_copy(src3_hbm.at[wid], src_v)
    pltpu.sync_copy(tgt3_hbm.at[wid], tgt_v)
    plsc.subcore_barrier()
    base0 = wid * EPW
    lane = lax.iota(jnp.int32, 16)
    gsems = (g0, g1)
    osems = (o0, o1)

    def issue(i, b):
        base = base0 + i * CH
        pltpu.async_copy(q_hbm.at[src_v.at[i]], qbuf.at[b], gsems[b])
        pltpu.async_copy(k_hbm.at[tgt_v.at[i]], kbuf.at[b], gsems[b])
        pltpu.async_copy(eattn_hbm.at[pl.ds(base, CH)], abuf.at[b], gsems[b])

    def wait_in(b):
        pltpu.make_async_copy(q_hbm.at[src_v.at[0]], qbuf.at[b], gsems[b]).wait()
        pltpu.make_async_copy(k_hbm.at[tgt_v.at[0]], kbuf.at[b], gsems[b]).wait()
        pltpu.make_async_copy(eattn_hbm.at[pl.ds(0, CH)], abuf.at[b], gsems[b]).wait()

    def compute(i, b):
        def edge_body(e, _):
            parts = []
            for h in range(NH):
                pv = (qbuf[b, e, pl.ds(16 * h, 16)]
                      * kbuf[b, e, pl.ds(16 * h, 16)]
                      * abuf[b, e, pl.ds(16 * h, 16)])
                parts.append(jnp.where(lane == h, jnp.sum(pv), 0.0))
            while len(parts) > 1:
                parts = [parts[i] + parts[i + 1]
                         for i in range(0, len(parts), 2)]
            ebuf[b, e, :] = jnp.exp(parts[0])
            return 0

        lax.fori_loop(0, CH, edge_body, 0, unroll=4)
        base = base0 + i * CH
        pltpu.async_copy(ebuf.at[b], expE_hbm.at[pl.ds(base, CH)], osems[b])
        pltpu.sync_copy(ebuf.at[b], accs.at[tgt_v.at[i]], add=True)

    def wait_out(b):
        pltpu.make_async_copy(ebuf.at[b], expE_hbm.at[pl.ds(0, CH)], osems[b]).wait()

    issue(0, 0)
    issue(1, 1)

    def pair_body(j, _):
        i0 = 2 * j
        wait_in(0)

        @pl.when(j > 0)
        def _():
            wait_out(0)

        compute(i0, 0)

        @pl.when(i0 + 2 < NCHUNK)
        def _():
            issue(i0 + 2, 0)

        wait_in(1)

        @pl.when(j > 0)
        def _():
            wait_out(1)

        compute(i0 + 1, 1)

        @pl.when(i0 + 3 < NCHUNK)
        def _():
            issue(i0 + 3, 1)

        return 0

    lax.fori_loop(0, NPAIR, pair_body, 0)
    wait_in(0)
    wait_out(0)
    compute(NCHUNK - 1, 0)
    wait_out(0)
    wait_out(1)
    plsc.subcore_barrier()

    @pl.when(s == 0)
    def _():
        pltpu.sync_copy(accs, spart_hbm.at[c])


@functools.partial(
    pl.kernel,
    out_type=jax.ShapeDtypeStruct((NC, N, H), F32),
    mesh=_mesh,
    compiler_params=_sc_params,
    scratch_types=[
        pltpu.VMEM((NCHUNK, CH), jnp.int32),
        pltpu.VMEM((2, CH, H), F32),
        pltpu.VMEM((2, CH, 16), F32),
        pltpu.VMEM_SHARED((N, H), F32),
        pltpu.SemaphoreType.DMA,
        pltpu.SemaphoreType.DMA,
        pltpu.SemaphoreType.DMA,
        pltpu.SemaphoreType.DMA,
    ],
)
def _sca2(le1_hbm, expE_hbm, tgt3_hbm, zm_hbm,
          mpart_hbm,
          tgt_v, lbuf, ebuf, accm,
          l0, l1, o0, o1):
    c = lax.axis_index("c")
    s = lax.axis_index("s")
    wid = s * NC + c

    @pl.when(s == 0)
    def _():
        pltpu.sync_copy(zm_hbm, accm)

    pltpu.sync_copy(tgt3_hbm.at[wid], tgt_v)
    plsc.subcore_barrier()
    base0 = wid * EPW
    lsems = (l0, l1)
    osems = (o0, o1)

    def issue(i, b):
        base = base0 + i * CH
        pltpu.async_copy(le1_hbm.at[pl.ds(base, CH)], lbuf.at[b], lsems[b])
        pltpu.async_copy(expE_hbm.at[pl.ds(base, CH)], ebuf.at[b], lsems[b])

    def wait_in(b):
        pltpu.make_async_copy(le1_hbm.at[pl.ds(0, CH)], lbuf.at[b], lsems[b]).wait()
        pltpu.make_async_copy(expE_hbm.at[pl.ds(0, CH)], ebuf.at[b], lsems[b]).wait()

    def compute(i, b):
        def edge_body(e, _):
            ev = ebuf[b, e, :]
            for h in range(NH):
                lbuf[b, e, pl.ds(16 * h, 16)] = (
                    lbuf[b, e, pl.ds(16 * h, 16)] * ev[h])
            return 0

        lax.fori_loop(0, CH, edge_body, 0, unroll=4)
        pltpu.sync_copy(lbuf.at[b], accm.at[tgt_v.at[i]], add=True)

    def wait_out(b):
        pass

    issue(0, 0)
    issue(1, 1)

    def pair_body(j, _):
        i0 = 2 * j
        wait_in(0)

        @pl.when(j > 0)
        def _():
            wait_out(0)

        compute(i0, 0)

        @pl.when(i0 + 2 < NCHUNK)
        def _():
            issue(i0 + 2, 0)

        wait_in(1)

        @pl.when(j > 0)
        def _():
            wait_out(1)

        compute(i0 + 1, 1)

        @pl.when(i0 + 3 < NCHUNK)
        def _():
            issue(i0 + 3, 1)

        return 0

    lax.fori_loop(0, NPAIR, pair_body, 0)
    wait_in(0)
    wait_out(0)
    compute(NCHUNK - 1, 0)
    wait_out(0)
    wait_out(1)
    plsc.subcore_barrier()

    @pl.when(s == 0)
    def _():
        pltpu.sync_copy(accm, mpart_hbm.at[c])


@functools.partial(
    pl.kernel,
    out_type=jax.ShapeDtypeStruct((N, H), F32),
    mesh=_mesh,
    compiler_params=_sc_params,
    scratch_types=[
        pltpu.VMEM((RN, H), F32),
        pltpu.VMEM((RN, H), F32),
        pltpu.VMEM((RN, H), F32),
        pltpu.VMEM((RN, 16), F32),
        pltpu.VMEM((RN, 16), F32),
        pltpu.SemaphoreType.DMA,
        pltpu.SemaphoreType.DMA,
        pltpu.SemaphoreType.DMA,
        pltpu.SemaphoreType.DMA,
    ],
)
def _scn(v_hbm, mpart_hbm, spart_hbm, out_hbm,
         m0buf, m1buf, vbuf, s0buf, s1buf, sem1, sem2, sem3, sem4):
    c = lax.axis_index("c")
    s = lax.axis_index("s")
    wid = s * NC + c

    def rchunk_body(i, _):
        cid = wid + i * NW

        @pl.when(cid < NRCHUNK)
        def _():
            base = cid * RN
            c0 = pltpu.async_copy(mpart_hbm.at[0].at[pl.ds(base, RN)], m0buf, sem1)
            c1 = pltpu.async_copy(mpart_hbm.at[1].at[pl.ds(base, RN)], m1buf, sem2)
            c2 = pltpu.async_copy(spart_hbm.at[0].at[pl.ds(base, RN)], s0buf, sem3)
            c3 = pltpu.async_copy(v_hbm.at[pl.ds(base, RN)], vbuf, sem4)
            pltpu.sync_copy(spart_hbm.at[1].at[pl.ds(base, RN)], s1buf)
            c0.wait()
            c1.wait()
            c2.wait()
            c3.wait()

            def row_body(r, _):
                sv = s0buf[r, :] + s1buf[r, :]
                rec = 1.0 / (sv + 1e-16)
                for h in range(NH):
                    m0buf[r, pl.ds(16 * h, 16)] = (
                        (m0buf[r, pl.ds(16 * h, 16)]
                         + m1buf[r, pl.ds(16 * h, 16)])
                        * vbuf[r, pl.ds(16 * h, 16)] * rec[h])
                return 0

            lax.fori_loop(0, RN, row_body, 0)
            pltpu.sync_copy(m0buf, out_hbm.at[pl.ds(base, RN)])

        return 0

    lax.fori_loop(0, RITER, rchunk_body, 0)


def _k3_body(x_ref, out_ref, mod_ref, g2_ref, b2_ref,
             w1_ref, w3_ref, w2_ref, wn2e_ref, bn2e_ref,
             hout_ref, ow_ref):
    out = out_ref[...]
    mod = mod_ref[...]
    h = x_ref[...] + mod[:, 256:384] * out
    h = (_ln(h) * g2_ref[...] + b2_ref[...]) * (1.0 + mod[:, 512:640]) + mod[:, 384:512]
    sw = _bdot(jax.nn.silu(_bdot(h, w1_ref)) * _bdot(h, w3_ref), w2_ref)
    hout_ref[...] = h + mod[:, 640:768] * sw
    ow_ref[...] = _bdot(out, wn2e_ref) + 0.5 * bn2e_ref[...]


def _k3(x, out, mod, g2, b2, W1, W3, W2, W_n2e, b_n2e):
    full = lambda a, b: pl.BlockSpec((a, b), lambda i: (0, 0))
    row = lambda w: pl.BlockSpec((NB, w), lambda i: (i, 0))
    return pl.pallas_call(
        _k3_body,
        grid=(N // NB,),
        in_specs=[row(H), row(H), row(6 * H),
                  full(1, H), full(1, H),
                  full(H, INNER), full(H, INNER), full(INNER, H),
                  full(H, H), full(1, H)],
        out_specs=[row(H), row(H)],
        out_shape=[jax.ShapeDtypeStruct((N, H), F32),
                   jax.ShapeDtypeStruct((N, H), F32)],
    )(x, out, mod, g2, b2, W1, W3, W2, W_n2e, b_n2e)


@functools.partial(
    pl.kernel,
    out_type=jax.ShapeDtypeStruct((E, H), F32),
    mesh=_mesh,
    compiler_params=_sc_params,
    scratch_types=[
        pltpu.VMEM((NCHUNK, CH), jnp.int32),
        pltpu.VMEM((NCHUNK, CH), jnp.int32),
        pltpu.VMEM((2, CH, H), F32),
        pltpu.VMEM((2, CH, H), F32),
        pltpu.SemaphoreType.DMA,
        pltpu.SemaphoreType.DMA,
        pltpu.SemaphoreType.DMA,
        pltpu.SemaphoreType.DMA,
    ],
)
def _sc3(ow_hbm, src3_hbm, tgt3_hbm, og_hbm,
         src_v, tgt_v, abuf, bbuf, g0, g1, o0, o1):
    c = lax.axis_index("c")
    s = lax.axis_index("s")
    wid = s * NC + c
    base0 = wid * EPW
    pltpu.sync_copy(src3_hbm.at[wid], src_v)
    pltpu.sync_copy(tgt3_hbm.at[wid], tgt_v)
    gsems = (g0, g1)
    osems = (o0, o1)

    def issue(i, b):
        pltpu.async_copy(ow_hbm.at[src_v.at[i]], abuf.at[b], gsems[b])
        pltpu.async_copy(ow_hbm.at[tgt_v.at[i]], bbuf.at[b], gsems[b])

    def wait_in(b):
        pltpu.make_async_copy(ow_hbm.at[src_v.at[0]], abuf.at[b], gsems[b]).wait()
        pltpu.make_async_copy(ow_hbm.at[tgt_v.at[0]], bbuf.at[b], gsems[b]).wait()

    def compute(i, b):
        def edge_body(e, _):
            for h in range(NH):
                abuf[b, e, pl.ds(16 * h, 16)] = (
                    abuf[b, e, pl.ds(16 * h, 16)]
                    + bbuf[b, e, pl.ds(16 * h, 16)])
            return 0

        lax.fori_loop(0, CH, edge_body, 0, unroll=4)
        base = base0 + i * CH
        pltpu.async_copy(abuf.at[b], og_hbm.at[pl.ds(base, CH)], osems[b])

    def wait_out(b):
        pltpu.make_async_copy(abuf.at[b], og_hbm.at[pl.ds(0, CH)], osems[b]).wait()

    issue(0, 0)
    issue(1, 1)

    def pair_body(j, _):
        i0 = 2 * j
        wait_in(0)

        @pl.when(j > 0)
        def _():
            wait_out(0)

        compute(i0, 0)

        @pl.when(i0 + 2 < NCHUNK)
        def _():
            issue(i0 + 2, 0)

        wait_in(1)

        @pl.when(j > 0)
        def _():
            wait_out(1)

        compute(i0 + 1, 1)

        @pl.when(i0 + 3 < NCHUNK)
        def _():
            issue(i0 + 3, 1)

        return 0

    lax.fori_loop(0, NPAIR, pair_body, 0)
    wait_in(0)
    wait_out(0)
    compute(NCHUNK - 1, 0)
    wait_out(0)
    wait_out(1)


def _k4_body(og_ref, eattr_ref, te_ref, ea_ref, wade_ref, bade_ref,
             we1_ref, we3_ref, we2_ref, out_ref):
    mod4 = _bdot(jax.nn.silu(te_ref[...]), wade_ref) + bade_ref[...]
    h = eattr_ref[...] + mod4[:, 0:128] * og_ref[...]
    h = _ln(h) * (1.0 + mod4[:, 256:384]) + mod4[:, 128:256]
    sw = _bdot(jax.nn.silu(_bdot(h, we1_ref)) * _bdot(h, we3_ref), we2_ref)
    out_ref[...] = ea_ref[...].astype(F32) + h + mod4[:, 384:512] * sw


def _k4(og, edge_attr, t_emb_e, ea, Wade4, bade4, We1, We3, We2):
    full = lambda a, b: pl.BlockSpec((a, b), lambda i: (0, 0))
    row = lambda w: pl.BlockSpec((EB, w), lambda i: (i, 0))
    return pl.pallas_call(
        _k4_body,
        grid=(E // EB,),
        in_specs=[row(H), row(H), row(H), row(H),
                  full(H, 4 * H), full(1, 4 * H),
                  full(H, INNER), full(H, INNER), full(INNER, H)],
        out_specs=row(H),
        out_shape=jax.ShapeDtypeStruct((E, H), F32),
    )(og, edge_attr, t_emb_e, ea, Wade4, bade4, We1, We3, We2)


def kernel(batch, x, t_emb_h, edge_attr, edge_index, t_emb_e, dist,
           W_edge_emb, b_edge_emb, W_ada, b_ada, W_ada_e, b_ada_e,
           W_qkv, W_le0, W_le1, W_n2e, b_n2e, g2, b2,
           W1, W3, W2, We1, We3, We2):
    src3 = edge_index[0].reshape(NW, NCHUNK, CH)
    tgt3 = edge_index[1].reshape(NW, NCHUNK, CH)

    bf = lambda w: w.astype(BF16)
    qkv, mod = _k1(x, t_emb_h, bf(W_ada), b_ada.reshape(1, -1), bf(W_qkv))
    Qn = qkv[:, 0:H]
    Kn = qkv[:, H:2 * H]
    Vn = qkv[:, 2 * H:3 * H]

    ea, e_attn, le1 = _k2(edge_attr, t_emb_e, dist,
                          bf(W_edge_emb[:H]), bf(W_edge_emb[H:]),
                          b_edge_emb.reshape(1, -1),
                          bf(W_ada_e[:, :2 * H]), b_ada_e[:2 * H].reshape(1, -1),
                          bf(W_le0), bf(W_le1))

    zs = jnp.zeros((N, 16), F32)
    expE, spart = _sca1(Qn, Kn, e_attn, src3, tgt3, zs)

    zm = jnp.zeros((N, H), F32)
    mpart = _sca2(le1, expE, tgt3, zm)

    out = _scn(Vn, mpart, spart)

    h_out, ow = _k3(x, out, mod,
                    g2.reshape(1, -1), b2.reshape(1, -1),
                    bf(W1), bf(W3), bf(W2), bf(W_n2e), b_n2e.reshape(1, -1))

    og = _sc3(ow, src3, tgt3)

    h_edge_out = _k4(og, edge_attr, t_emb_e, ea,
                     bf(W_ada_e[:, 2 * H:]), b_ada_e[2 * H:].reshape(1, -1),
                     bf(We1), bf(We3), bf(We2))

    return (h_out, h_edge_out)


# speedup vs baseline: 1.1056x; 1.0159x over previous
import functools

import jax
import jax.numpy as jnp
from jax import lax
from jax.experimental import pallas as pl
from jax.experimental.pallas import tpu as pltpu
from jax.experimental.pallas import tpu_sc as plsc

F32 = jnp.float32
BF16 = jnp.bfloat16

N = 10000
E = 320000
H = 128
NH = 8
DH = 16
INNER = 512

NB = 1000
EB = 3200
NC = 2
NS = 16
NW = NC * NS
EPW = E // NW
CH = 80
NCHUNK = EPW // CH
NPAIR = NCHUNK // 2

RN = 16
NRCHUNK = N // RN
RITER = (NRCHUNK + NW - 1) // NW

_mesh = plsc.VectorSubcoreMesh(core_axis_name="c", subcore_axis_name="s")
_sc_params = pltpu.CompilerParams(needs_layout_passes=False,
                                  use_tc_tiling_on_sc=False)


def _bdot(a, w_ref):
    return jnp.dot(a.astype(BF16), w_ref[...], preferred_element_type=F32)


def _ln(v):
    mu = jnp.mean(v, axis=-1, keepdims=True)
    var = jnp.mean((v - mu) ** 2, axis=-1, keepdims=True)
    return (v - mu) * lax.rsqrt(var + 1e-6)


def _k1_body(x_ref, te_ref, wada_ref, bada_ref, wqkv_ref, qkv_ref, mod_ref):
    mod = _bdot(jax.nn.silu(te_ref[...]), wada_ref) + bada_ref[...]
    mod_ref[...] = mod
    xm = _ln(x_ref[...]) * (1.0 + mod[:, 128:256]) + mod[:, 0:128]
    qkv_ref[...] = _bdot(xm, wqkv_ref)


def _k1(x, t_emb_h, W_ada, b_ada, W_qkv):
    return pl.pallas_call(
        _k1_body,
        grid=(N // NB,),
        in_specs=[
            pl.BlockSpec((NB, H), lambda i: (i, 0)),
            pl.BlockSpec((NB, H), lambda i: (i, 0)),
            pl.BlockSpec((H, 6 * H), lambda i: (0, 0)),
            pl.BlockSpec((1, 6 * H), lambda i: (0, 0)),
            pl.BlockSpec((H, 3 * H), lambda i: (0, 0)),
        ],
        out_specs=[
            pl.BlockSpec((NB, 3 * H), lambda i: (i, 0)),
            pl.BlockSpec((NB, 6 * H), lambda i: (i, 0)),
        ],
        out_shape=[
            jax.ShapeDtypeStruct((N, 3 * H), F32),
            jax.ShapeDtypeStruct((N, 6 * H), F32),
        ],
    )(x, t_emb_h, W_ada, b_ada, W_qkv)


def _k2_body(eattr_ref, te_ref, dist_ref, wee1_ref, wee2_ref, bee_ref,
             wade_ref, bade_ref, wle0_ref, wle1_ref,
             ea_ref, eattn_ref, le1_ref):
    mod2 = _bdot(jax.nn.silu(te_ref[...]), wade_ref) + bade_ref[...]
    ea = (_bdot(eattr_ref[...], wee1_ref) + _bdot(dist_ref[...], wee2_ref)
          + bee_ref[...])
    ea_ref[...] = ea.astype(BF16)
    eam = _ln(ea) * (1.0 + mod2[:, 128:256]) + mod2[:, 0:128]
    eattn_ref[...] = jax.nn.gelu(_bdot(eam, wle0_ref), approximate=True) * 0.25
    le1_ref[...] = _bdot(eam, wle1_ref)


def _k2(edge_attr, t_emb_e, dist, Wee1, Wee2, bee, Wade2, bade2, W_le0, W_le1):
    full = lambda a, b: pl.BlockSpec((a, b), lambda i: (0, 0))
    row = lambda w: pl.BlockSpec((EB, w), lambda i: (i, 0))
    return pl.pallas_call(
        _k2_body,
        grid=(E // EB,),
        in_specs=[row(H), row(H), row(H),
                  full(H, H), full(H, H), full(1, H),
                  full(H, 2 * H), full(1, 2 * H), full(H, H), full(H, H)],
        out_specs=[row(H), row(H), row(H)],
        out_shape=[jax.ShapeDtypeStruct((E, H), BF16),
                   jax.ShapeDtypeStruct((E, H), F32),
                   jax.ShapeDtypeStruct((E, H), F32)],
    )(edge_attr, t_emb_e, dist, Wee1, Wee2, bee, Wade2, bade2, W_le0, W_le1)


@functools.partial(
    pl.kernel,
    out_type=(jax.ShapeDtypeStruct((E, 16), F32),
              jax.ShapeDtypeStruct((NC, N, 16), F32)),
    mesh=_mesh,
    compiler_params=_sc_params,
    scratch_types=[
        pltpu.VMEM((NCHUNK, CH), jnp.int32),
        pltpu.VMEM((NCHUNK, CH), jnp.int32),
        pltpu.VMEM((2, CH, H), F32),
        pltpu.VMEM((2, CH, H), F32),
        pltpu.VMEM((2, CH, H), F32),
        pltpu.VMEM((2, CH, 16), F32),
        pltpu.VMEM_SHARED((N, 16), F32),
        pltpu.SemaphoreType.DMA,
        pltpu.SemaphoreType.DMA,
        pltpu.SemaphoreType.DMA,
        pltpu.SemaphoreType.DMA,
    ],
)
def _sca1(q_hbm, k_hbm, eattn_hbm, src3_hbm, tgt3_hbm, zs_hbm,
          expE_hbm, spart_hbm,
          src_v, tgt_v, qbuf, kbuf, abuf, ebuf, accs,
          g0, g1, o0, o1):
    c = lax.axis_index("c")
    s = lax.axis_index("s")
    wid = s * NC + c

    @pl.when(s == 0)
    def _():
        pltpu.sync_copy(zs_hbm, accs)

    pltpu.sync_copy(src3_hbm.at[wid], src_v)
    pltpu.sync_copy(tgt3_hbm.at[wid], tgt_v)
    plsc.subcore_barrier()
    base0 = wid * EPW
    lane = lax.iota(jnp.int32, 16)
    gsems = (g0, g1)
    osems = (o0, o1)

    def issue(i, b):
        base = base0 + i * CH
        pltpu.async_copy(q_hbm.at[src_v.at[i]], qbuf.at[b], gsems[b])
        pltpu.async_copy(k_hbm.at[tgt_v.at[i]], kbuf.at[b], gsems[b])
        pltpu.async_copy(eattn_hbm.at[pl.ds(base, CH)], abuf.at[b], gsems[b])

    def wait_in(b):
        pltpu.make_async_copy(q_hbm.at[src_v.at[0]], qbuf.at[b], gsems[b]).wait()
        pltpu.make_async_copy(k_hbm.at[tgt_v.at[0]], kbuf.at[b], gsems[b]).wait()
        pltpu.make_async_copy(eattn_hbm.at[pl.ds(0, CH)], abuf.at[b], gsems[b]).wait()

    def compute(i, b):
        def edge_body(e, _):
            parts = []
            for h in range(NH):
                pv = (qbuf[b, e, pl.ds(16 * h, 16)]
                      * kbuf[b, e, pl.ds(16 * h, 16)]
                      * abuf[b, e, pl.ds(16 * h, 16)])
                parts.append(jnp.where(lane == h, jnp.sum(pv), 0.0))
            while len(parts) > 1:
                parts = [parts[i] + parts[i + 1]
                         for i in range(0, len(parts), 2)]
            ebuf[b, e, :] = jnp.exp(parts[0])
            return 0

        lax.fori_loop(0, CH, edge_body, 0, unroll=4)
        base = base0 + i * CH
        pltpu.async_copy(ebuf.at[b], expE_hbm.at[pl.ds(base, CH)], osems[b])
        pltpu.sync_copy(ebuf.at[b], accs.at[tgt_v.at[i]], add=True)

    def wait_out(b):
        pltpu.make_async_copy(ebuf.at[b], expE_hbm.at[pl.ds(0, CH)], osems[b]).wait()

    issue(0, 0)
    issue(1, 1)

    def pair_body(j, _):
        i0 = 2 * j
        wait_in(0)

        @pl.when(j > 0)
        def _():
            wait_out(0)

        compute(i0, 0)

        @pl.when(i0 + 2 < NCHUNK)
        def _():
            issue(i0 + 2, 0)

        wait_in(1)

        @pl.when(j > 0)
        def _():
            wait_out(1)

        compute(i0 + 1, 1)

        @pl.when(i0 + 3 < NCHUNK)
        def _():
            issue(i0 + 3, 1)

        return 0

    lax.fori_loop(0, NPAIR, pair_body, 0)
    wait_in(0)
    wait_out(0)
    compute(NCHUNK - 1, 0)
    wait_out(0)
    wait_out(1)
    plsc.subcore_barrier()

    @pl.when(s == 0)
    def _():
        pltpu.sync_copy(accs, spart_hbm.at[c])


@functools.partial(
    pl.kernel,
    out_type=jax.ShapeDtypeStruct((NC, N, H), F32),
    mesh=_mesh,
    compiler_params=_sc_params,
    scratch_types=[
        pltpu.VMEM((NCHUNK, CH), jnp.int32),
        pltpu.VMEM((2, CH, H), F32),
        pltpu.VMEM((2, CH, 16), F32),
        pltpu.VMEM_SHARED((N, H), F32),
        pltpu.SemaphoreType.DMA,
        pltpu.SemaphoreType.DMA,
        pltpu.SemaphoreType.DMA,
        pltpu.SemaphoreType.DMA,
    ],
)
def _sca2(le1_hbm, expE_hbm, tgt3_hbm, zm_hbm,
          mpart_hbm,
          tgt_v, lbuf, ebuf, accm,
          l0, l1, o0, o1):
    c = lax.axis_index("c")
    s = lax.axis_index("s")
    wid = s * NC + c

    @pl.when(s == 0)
    def _():
        pltpu.sync_copy(zm_hbm, accm)

    pltpu.sync_copy(tgt3_hbm.at[wid], tgt_v)
    plsc.subcore_barrier()
    base0 = wid * EPW
    lsems = (l0, l1)
    osems = (o0, o1)

    def issue(i, b):
        base = base0 + i * CH
        pltpu.async_copy(le1_hbm.at[pl.ds(base, CH)], lbuf.at[b], lsems[b])
        pltpu.async_copy(expE_hbm.at[pl.ds(base, CH)], ebuf.at[b], lsems[b])

    def wait_in(b):
        pltpu.make_async_copy(le1_hbm.at[pl.ds(0, CH)], lbuf.at[b], lsems[b]).wait()
        pltpu.make_async_copy(expE_hbm.at[pl.ds(0, CH)], ebuf.at[b], lsems[b]).wait()

    def compute(i, b):
        def edge_body(e, _):
            ev = ebuf[b, e, :]
            for h in range(NH):
                lbuf[b, e, pl.ds(16 * h, 16)] = (
                    lbuf[b, e, pl.ds(16 * h, 16)] * ev[h])
            return 0

        lax.fori_loop(0, CH, edge_body, 0, unroll=4)
        pltpu.sync_copy(lbuf.at[b], accm.at[tgt_v.at[i]], add=True)

    def wait_out(b):
        pass

    issue(0, 0)
    issue(1, 1)

    def pair_body(j, _):
        i0 = 2 * j
        wait_in(0)

        @pl.when(j > 0)
        def _():
            wait_out(0)

        compute(i0, 0)

        @pl.when(i0 + 2 < NCHUNK)
        def _():
            issue(i0 + 2, 0)

        wait_in(1)

        @pl.when(j > 0)
        def _():
            wait_out(1)

        compute(i0 + 1, 1)

        @pl.when(i0 + 3 < NCHUNK)
        def _():
            issue(i0 + 3, 1)

        return 0

    lax.fori_loop(0, NPAIR, pair_body, 0)
    wait_in(0)
    wait_out(0)
    compute(NCHUNK - 1, 0)
    wait_out(0)
    wait_out(1)
    plsc.subcore_barrier()

    @pl.when(s == 0)
    def _():
        pltpu.sync_copy(accm, mpart_hbm.at[c])


@functools.partial(
    pl.kernel,
    out_type=jax.ShapeDtypeStruct((N, H), F32),
    mesh=_mesh,
    compiler_params=_sc_params,
    scratch_types=[
        pltpu.VMEM((RN, H), F32),
        pltpu.VMEM((RN, H), F32),
        pltpu.VMEM((RN, H), F32),
        pltpu.VMEM((RN, 16), F32),
        pltpu.VMEM((RN, 16), F32),
        pltpu.SemaphoreType.DMA,
        pltpu.SemaphoreType.DMA,
        pltpu.SemaphoreType.DMA,
        pltpu.SemaphoreType.DMA,
    ],
)
def _scn(v_hbm, mpart_hbm, spart_hbm, out_hbm,
         m0buf, m1buf, vbuf, s0buf, s1buf, sem1, sem2, sem3, sem4):
    c = lax.axis_index("c")
    s = lax.axis_index("s")
    wid = s * NC + c

    def rchunk_body(i, _):
        cid = wid + i * NW

        @pl.when(cid < NRCHUNK)
        def _():
            base = cid * RN
            c0 = pltpu.async_copy(mpart_hbm.at[0].at[pl.ds(base, RN)], m0buf, sem1)
            c1 = pltpu.async_copy(mpart_hbm.at[1].at[pl.ds(base, RN)], m1buf, sem2)
            c2 = pltpu.async_copy(spart_hbm.at[0].at[pl.ds(base, RN)], s0buf, sem3)
            c3 = pltpu.async_copy(v_hbm.at[pl.ds(base, RN)], vbuf, sem4)
            pltpu.sync_copy(spart_hbm.at[1].at[pl.ds(base, RN)], s1buf)
            c0.wait()
            c1.wait()
            c2.wait()
            c3.wait()

            def row_body(r, _):
                sv = s0buf[r, :] + s1buf[r, :]
                rec = 1.0 / (sv + 1e-16)
                for h in range(NH):
                    m0buf[r, pl.ds(16 * h, 16)] = (
                        (m0buf[r, pl.ds(16 * h, 16)]
                         + m1buf[r, pl.ds(16 * h, 16)])
                        * vbuf[r, pl.ds(16 * h, 16)] * rec[h])
                return 0

            lax.fori_loop(0, RN, row_body, 0)
            pltpu.sync_copy(m0buf, out_hbm.at[pl.ds(base, RN)])

        return 0

    lax.fori_loop(0, RITER, rchunk_body, 0)


def _k3_body(x_ref, out_ref, mod_ref, g2_ref, b2_ref,
             w1_ref, w3_ref, w2_ref, wn2e_ref, bn2e_ref,
             hout_ref, ow_ref):
    out = out_ref[...]
    mod = mod_ref[...]
    h = x_ref[...] + mod[:, 256:384] * out
    h = (_ln(h) * g2_ref[...] + b2_ref[...]) * (1.0 + mod[:, 512:640]) + mod[:, 384:512]
    sw = _bdot(jax.nn.silu(_bdot(h, w1_ref)) * _bdot(h, w3_ref), w2_ref)
    hout_ref[...] = h + mod[:, 640:768] * sw
    ow_ref[...] = _bdot(out, wn2e_ref) + 0.5 * bn2e_ref[...]


def _k3(x, out, mod, g2, b2, W1, W3, W2, W_n2e, b_n2e):
    full = lambda a, b: pl.BlockSpec((a, b), lambda i: (0, 0))
    row = lambda w: pl.BlockSpec((NB, w), lambda i: (i, 0))
    return pl.pallas_call(
        _k3_body,
        grid=(N // NB,),
        in_specs=[row(H), row(H), row(6 * H),
                  full(1, H), full(1, H),
                  full(H, INNER), full(H, INNER), full(INNER, H),
                  full(H, H), full(1, H)],
        out_specs=[row(H), row(H)],
        out_shape=[jax.ShapeDtypeStruct((N, H), F32),
                   jax.ShapeDtypeStruct((N, H), F32)],
    )(x, out, mod, g2, b2, W1, W3, W2, W_n2e, b_n2e)


@functools.partial(
    pl.kernel,
    out_type=jax.ShapeDtypeStruct((E, H), F32),
    mesh=_mesh,
    compiler_params=_sc_params,
    scratch_types=[
        pltpu.VMEM((NCHUNK, CH), jnp.int32),
        pltpu.VMEM((NCHUNK, CH), jnp.int32),
        pltpu.VMEM((2, CH, H), F32),
        pltpu.VMEM((2, CH, H), F32),
        pltpu.SemaphoreType.DMA,
        pltpu.SemaphoreType.DMA,
        pltpu.SemaphoreType.DMA,
        pltpu.SemaphoreType.DMA,
    ],
)
def _sc3(ow_hbm, src3_hbm, tgt3_hbm, og_hbm,
         src_v, tgt_v, abuf, bbuf, g0, g1, o0, o1):
    c = lax.axis_index("c")
    s = lax.axis_index("s")
    wid = s * NC + c
    base0 = wid * EPW
    pltpu.sync_copy(src3_hbm.at[wid], src_v)
    pltpu.sync_copy(tgt3_hbm.at[wid], tgt_v)
    gsems = (g0, g1)
    osems = (o0, o1)

    def issue(i, b):
        pltpu.async_copy(ow_hbm.at[src_v.at[i]], abuf.at[b], gsems[b])
        pltpu.async_copy(ow_hbm.at[tgt_v.at[i]], bbuf.at[b], gsems[b])

    def wait_in(b):
        pltpu.make_async_copy(ow_hbm.at[src_v.at[0]], abuf.at[b], gsems[b]).wait()
        pltpu.make_async_copy(ow_hbm.at[tgt_v.at[0]], bbuf.at[b], gsems[b]).wait()

    def compute(i, b):
        def edge_body(e, _):
            for h in range(NH):
                abuf[b, e, pl.ds(16 * h, 16)] = (
                    abuf[b, e, pl.ds(16 * h, 16)]
                    + bbuf[b, e, pl.ds(16 * h, 16)])
            return 0

        lax.fori_loop(0, CH, edge_body, 0, unroll=4)
        base = base0 + i * CH
        pltpu.async_copy(abuf.at[b], og_hbm.at[pl.ds(base, CH)], osems[b])

    def wait_out(b):
        pltpu.make_async_copy(abuf.at[b], og_hbm.at[pl.ds(0, CH)], osems[b]).wait()

    issue(0, 0)
    issue(1, 1)

    def pair_body(j, _):
        i0 = 2 * j
        wait_in(0)

        @pl.when(j > 0)
        def _():
            wait_out(0)

        compute(i0, 0)

        @pl.when(i0 + 2 < NCHUNK)
        def _():
            issue(i0 + 2, 0)

        wait_in(1)

        @pl.when(j > 0)
        def _():
            wait_out(1)

        compute(i0 + 1, 1)

        @pl.when(i0 + 3 < NCHUNK)
        def _():
            issue(i0 + 3, 1)

        return 0

    lax.fori_loop(0, NPAIR, pair_body, 0)
    wait_in(0)
    wait_out(0)
    compute(NCHUNK - 1, 0)
    wait_out(0)
    wait_out(1)


def _k4_body(og_ref, eattr_ref, te_ref, ea_ref, wade_ref, bade_ref,
             we1_ref, we3_ref, we2_ref, out_ref):
    mod4 = _bdot(jax.nn.silu(te_ref[...]), wade_ref) + bade_ref[...]
    h = eattr_ref[...] + mod4[:, 0:128] * og_ref[...]
    h = _ln(h) * (1.0 + mod4[:, 256:384]) + mod4[:, 128:256]
    sw = _bdot(jax.nn.silu(_bdot(h, we1_ref)) * _bdot(h, we3_ref), we2_ref)
    out_ref[...] = ea_ref[...].astype(F32) + h + mod4[:, 384:512] * sw


def _k4(og, edge_attr, t_emb_e, ea, Wade4, bade4, We1, We3, We2):
    full = lambda a, b: pl.BlockSpec((a, b), lambda i: (0, 0))
    row = lambda w: pl.BlockSpec((EB, w), lambda i: (i, 0))
    return pl.pallas_call(
        _k4_body,
        grid=(E // EB,),
        in_specs=[row(H), row(H), row(H), row(H),
                  full(H, 4 * H), full(1, 4 * H),
                  full(H, INNER), full(H, INNER), full(INNER, H)],
        out_specs=row(H),
        out_shape=jax.ShapeDtypeStruct((E, H), F32),
    )(og, edge_attr, t_emb_e, ea, Wade4, bade4, We1, We3, We2)


def kernel(batch, x, t_emb_h, edge_attr, edge_index, t_emb_e, dist,
           W_edge_emb, b_edge_emb, W_ada, b_ada, W_ada_e, b_ada_e,
           W_qkv, W_le0, W_le1, W_n2e, b_n2e, g2, b2,
           W1, W3, W2, We1, We3, We2):
    src3 = edge_index[0].reshape(NW, NCHUNK, CH)
    tgt3 = edge_index[1].reshape(NW, NCHUNK, CH)

    bf = lambda w: w.astype(BF16)
    qkv, mod = _k1(x, t_emb_h, bf(W_ada), b_ada.reshape(1, -1), bf(W_qkv))
    Qn = qkv[:, 0:H]
    Kn = qkv[:, H:2 * H]
    Vn = qkv[:, 2 * H:3 * H]

    ea, e_attn, le1 = _k2(edge_attr, t_emb_e, dist,
                          bf(W_edge_emb[:H]), bf(W_edge_emb[H:]),
                          b_edge_emb.reshape(1, -1),
                          bf(W_ada_e[:, :2 * H]), b_ada_e[:2 * H].reshape(1, -1),
                          bf(W_le0), bf(W_le1))

    zs = jnp.zeros((N, 16), F32)
    expE, spart = _sca1(Qn, Kn, e_attn, src3, tgt3, zs)

    zm = jnp.zeros((N, H), F32)
    mpart = _sca2(le1, expE, tgt3, zm)

    out = _scn(Vn, mpart, spart)

    h_out, ow = _k3(x, out, mod,
                    g2.reshape(1, -1), b2.reshape(1, -1),
                    bf(W1), bf(W3), bf(W2), bf(W_n2e), b_n2e.reshape(1, -1))

    og = _sc3(ow, src3, tgt3)

    h_edge_out = _k4(og, edge_attr, t_emb_e, ea,
                     bf(W_ada_e[:, 2 * H:]), b_ada_e[2 * H:].reshape(1, -1),
                     bf(We1), bf(We3), bf(We2))

    return (h_out, h_edge_out)
